# Initial kernel scaffold; baseline (speedup 1.0000x reference)
import jax
import jax.numpy as jnp
from jax.experimental import pallas as pl


def kernel(x, edge_index, edge_attr, lin_in_w, lin_in_b, enn_w, enn_b, root_w, root_b, bn_g, bn_b, cls_w, cls_b):
    raise NotImplementedError("write your pallas kernel here")


# speedup vs baseline: 7.5992x; 7.5992x over previous
import functools

import jax
import jax.numpy as jnp
from jax import lax
from jax.experimental import pallas as pl
from jax.experimental.pallas import tpu as pltpu
from jax.experimental.pallas import tpu_sc as plsc

N = 10000
E = 320000
D_IN = 128
H = 16
D_E = 4
L = 3
OUT = 2

NC = 2
NS = 16
NW = NC * NS
EW = E // NW
CHUNK = 1000
NCHUNK = EW // CHUNK
UROW = (D_E + 1) * H
NPAD = 10240
ROWS_PER_TILE = NPAD // NS


def _in_proj_kernel(x_ref, w_ref, b_ref, wcat_ref, rw_ref, rb_ref,
                    u_ref, hr_ref):
    h0 = jnp.dot(x_ref[...], w_ref[...],
                 preferred_element_type=jnp.float32) + b_ref[...]
    u_ref[...] = jnp.dot(h0, wcat_ref[...], preferred_element_type=jnp.float32)
    hr_ref[...] = jnp.dot(h0, rw_ref[...],
                          preferred_element_type=jnp.float32) + rb_ref[...]


def _mid_kernel(hr_ref, part_ref, g_ref, b_ref, wcat_ref, rw_ref, rb_ref,
                u_ref, hro_ref):
    t = (hr_ref[...] + part_ref[0, pl.ds(0, N), :]
         + part_ref[1, pl.ds(0, N), :])
    mean = jnp.mean(t, axis=0, keepdims=True)
    var = jnp.mean((t - mean) ** 2, axis=0, keepdims=True)
    hn = (t - mean) * lax.rsqrt(var + 1e-5) * g_ref[...] + b_ref[...]
    hn = jnp.maximum(hn, 0.0)
    u_ref[...] = jnp.dot(hn, wcat_ref[...], preferred_element_type=jnp.float32)
    hro_ref[...] = jnp.dot(hn, rw_ref[...],
                           preferred_element_type=jnp.float32) + rb_ref[...]


def _final_kernel(hr_ref, part_ref, g_ref, b_ref, cw_ref, cb_ref, o_ref):
    t = (hr_ref[...] + part_ref[0, pl.ds(0, N), :]
         + part_ref[1, pl.ds(0, N), :])
    mean = jnp.mean(t, axis=0, keepdims=True)
    var = jnp.mean((t - mean) ** 2, axis=0, keepdims=True)
    hn = (t - mean) * lax.rsqrt(var + 1e-5) * g_ref[...] + b_ref[...]
    hn = jnp.maximum(hn, 0.0)
    o_ref[...] = jnp.dot(hn, cw_ref[...],
                         preferred_element_type=jnp.float32) + cb_ref[...]


_in_proj = pl.pallas_call(
    _in_proj_kernel,
    out_shape=(jax.ShapeDtypeStruct((N, UROW), jnp.float32),
               jax.ShapeDtypeStruct((N, H), jnp.float32)),
)

_mid = pl.pallas_call(
    _mid_kernel,
    out_shape=(jax.ShapeDtypeStruct((N, UROW), jnp.float32),
               jax.ShapeDtypeStruct((N, H), jnp.float32)),
)

_final = pl.pallas_call(
    _final_kernel,
    out_shape=jax.ShapeDtypeStruct((N, OUT), jnp.float32),
)


def _edge_body(u_hbm, src_hbm, dst_hbm, ea_hbm, out_hbm,
               srcv, dstv, eav, rows, msg, buf, agg, sem):
    cid = lax.axis_index("c")
    sid = lax.axis_index("s")

    zero = jnp.zeros((H,), jnp.float32)

    def _zbody(i, c):
        buf[i, :] = zero
        return c
    lax.fori_loop(0, ROWS_PER_TILE, _zbody, 0, unroll=8)

    pltpu.sync_copy(buf, agg.at[pl.ds(sid * ROWS_PER_TILE, ROWS_PER_TILE)])
    plsc.subcore_barrier()

    wid = sid * NC + cid
    for k in range(NCHUNK):
        base = wid * EW + k * CHUNK
        pltpu.sync_copy(src_hbm.at[pl.ds(base, CHUNK)], srcv)
        pltpu.sync_copy(dst_hbm.at[pl.ds(base, CHUNK)], dstv)
        pltpu.sync_copy(ea_hbm.at[pl.ds(base * D_E, CHUNK * D_E)],
                        eav.at[pl.ds(0, CHUNK * D_E)])
        pltpu.async_copy(u_hbm.at[srcv], rows, sem).wait()

        def _ebody(e, c):
            a = eav[pl.ds(D_E * e, H)]
            m = rows[e, pl.ds(4 * H, H)]
            m = m + a[0] * rows[e, pl.ds(0, H)]
            m = m + a[1] * rows[e, pl.ds(H, H)]
            m = m + a[2] * rows[e, pl.ds(2 * H, H)]
            m = m + a[3] * rows[e, pl.ds(3 * H, H)]
            msg[e, :] = m
            return c
        lax.fori_loop(0, CHUNK, _ebody, 0, unroll=4)

        pltpu.sync_copy(msg, agg.at[dstv], add=True)

    plsc.subcore_barrier()
    pltpu.sync_copy(agg.at[pl.ds(sid * ROWS_PER_TILE, ROWS_PER_TILE)], buf)
    pltpu.sync_copy(buf,
                    out_hbm.at[cid, pl.ds(sid * ROWS_PER_TILE,
                                          ROWS_PER_TILE)])


_edge_pass = pl.kernel(
    _edge_body,
    out_type=jax.ShapeDtypeStruct((NC, NPAD, H), jnp.float32),
    mesh=plsc.VectorSubcoreMesh(core_axis_name="c", subcore_axis_name="s",
                                num_cores=NC, num_subcores=NS),
    compiler_params=pltpu.CompilerParams(use_tc_tiling_on_sc=False),
    scratch_types=[
        pltpu.VMEM((CHUNK,), jnp.int32),
        pltpu.VMEM((CHUNK,), jnp.int32),
        pltpu.VMEM((CHUNK * D_E + H,), jnp.float32),
        pltpu.VMEM((CHUNK, UROW), jnp.float32),
        pltpu.VMEM((CHUNK, H), jnp.float32),
        pltpu.VMEM((ROWS_PER_TILE, H), jnp.float32),
        pltpu.VMEM_SHARED((NPAD, H), jnp.float32),
        pltpu.SemaphoreType.DMA,
    ],
)


def kernel(x, edge_index, edge_attr, lin_in_w, lin_in_b, enn_w, enn_b,
           root_w, root_b, bn_g, bn_b, cls_w, cls_b):
    src = edge_index[0]
    dst = edge_index[1]
    ea_flat = edge_attr.reshape(-1)

    wcat = jnp.concatenate(
        [enn_w.reshape(L, D_E, H, H).transpose(0, 2, 1, 3).reshape(L, H, D_E * H),
         enn_b.reshape(L, H, H)], axis=2)

    u, hr = _in_proj(x, lin_in_w, lin_in_b.reshape(1, H),
                     wcat[0], root_w[0], root_b[0].reshape(1, H))
    for l in range(L):
        part = _edge_pass(u, src, dst, ea_flat)
        if l + 1 < L:
            u, hr = _mid(hr, part, bn_g[l].reshape(1, H), bn_b[l].reshape(1, H),
                         wcat[l + 1], root_w[l + 1], root_b[l + 1].reshape(1, H))
        else:
            out = _final(hr, part, bn_g[l].reshape(1, H), bn_b[l].reshape(1, H),
                         cls_w, cls_b.reshape(1, OUT))
    return out
